# baseline (speedup 1.0000x reference)
import functools

import jax
import jax.numpy as jnp
from jax.experimental import pallas as pl
from jax.experimental.pallas import tpu as pltpu

B, N, M = 4, 4096, 4096
BI = 512
NB = N // BI


def _chamfer_body(x1_ref, y1_ref, x2_ref, y2_ref, out_ref, colmin_ref):
    b = pl.program_id(0)
    ib = pl.program_id(1)

    x1 = x1_ref[0, 0, :].reshape(BI, 1)
    y1 = y1_ref[0, 0, :].reshape(BI, 1)
    x2 = x2_ref[0, 0, :].reshape(1, M)
    y2 = y2_ref[0, 0, :].reshape(1, M)

    bx1 = x1.astype(jnp.bfloat16).astype(jnp.float32)
    by1 = y1.astype(jnp.bfloat16).astype(jnp.float32)
    bx2 = x2.astype(jnp.bfloat16).astype(jnp.float32)
    by2 = y2.astype(jnp.bfloat16).astype(jnp.float32)
    ab = bx1 * bx2 + by1 * by2
    a2 = x1 * x1 + y1 * y1
    b2 = x2 * x2 + y2 * y2
    d = jnp.maximum(a2 + b2 - 2.0 * ab, 0.0)

    rowmin = jnp.min(d, axis=1)
    colmin = jnp.min(d, axis=0).reshape(1, M)

    @pl.when(ib == 0)
    def _init_col():
        colmin_ref[...] = colmin

    @pl.when(ib != 0)
    def _acc_col():
        colmin_ref[...] = jnp.minimum(colmin_ref[...], colmin)

    @pl.when(jnp.logical_and(b == 0, ib == 0))
    def _init_out():
        out_ref[0, 0] = 0.0

    partial = jnp.sum(rowmin) * (1.0 / N)

    @pl.when(ib == NB - 1)
    def _finish_batch():
        out_ref[0, 0] += partial + jnp.sum(colmin_ref[...]) * (1.0 / M)

    @pl.when(ib != NB - 1)
    def _acc_row():
        out_ref[0, 0] += partial


@jax.jit
def kernel(points1, points2):
    x1 = points1[..., 0].reshape(B * NB, 1, BI)
    y1 = points1[..., 1].reshape(B * NB, 1, BI)
    x2 = points2[..., 0].reshape(B, 1, M)
    y2 = points2[..., 1].reshape(B, 1, M)

    out = pl.pallas_call(
        _chamfer_body,
        grid=(B, NB),
        in_specs=[
            pl.BlockSpec((1, 1, BI), lambda b, i: (b * NB + i, 0, 0)),
            pl.BlockSpec((1, 1, BI), lambda b, i: (b * NB + i, 0, 0)),
            pl.BlockSpec((1, 1, M), lambda b, i: (b, 0, 0)),
            pl.BlockSpec((1, 1, M), lambda b, i: (b, 0, 0)),
        ],
        out_specs=pl.BlockSpec(
            (1, 1), lambda b, i: (0, 0), memory_space=pltpu.SMEM
        ),
        out_shape=jax.ShapeDtypeStruct((1, 1), jnp.float32),
        scratch_shapes=[pltpu.VMEM((1, M), jnp.float32)],
    )(x1, y1, x2, y2)
    return out[0, 0]
